# baseline (speedup 1.0000x reference)
import functools

import jax
import jax.numpy as jnp
from jax import lax
from jax.experimental import pallas as pl
from jax.experimental.pallas import tpu as pltpu
from jax.experimental.pallas import tpu_sc as plsc

_NUM_CORES = 2
_NUM_SUBCORES = 16
_NW = _NUM_CORES * _NUM_SUBCORES
_CHUNK = 128


@functools.lru_cache(maxsize=None)
def _make_gather(b: int, d: int):
  assert b % (_NW * _CHUNK) == 0
  b_per_w = b // _NW
  n_chunks = b_per_w // _CHUNK
  mesh = plsc.VectorSubcoreMesh(core_axis_name="c", subcore_axis_name="s")

  @functools.partial(
      pl.kernel,
      mesh=mesh,
      out_type=jax.ShapeDtypeStruct((b, d), jnp.float32),
      compiler_params=pltpu.CompilerParams(use_tc_tiling_on_sc=False),
      scratch_types=[
          pltpu.VMEM((b_per_w,), jnp.int32),
          pltpu.VMEM((2, _CHUNK, d), jnp.float32),
          pltpu.SemaphoreType.DMA,
          pltpu.SemaphoreType.DMA,
          pltpu.SemaphoreType.DMA,
          pltpu.SemaphoreType.DMA,
      ],
  )
  def gather_kernel(table_hbm, idx_hbm, out_hbm, idx_v, rows_v, gsem0,
                    gsem1, osem0, osem1):
    wid = lax.axis_index("s") * _NUM_CORES + lax.axis_index("c")
    base = wid * b_per_w
    gsems = (gsem0, gsem1)
    osems = (osem0, osem1)

    pltpu.sync_copy(idx_hbm.at[pl.ds(base, b_per_w)], idx_v)

    def start_gather(c, buf):
      pltpu.async_copy(
          table_hbm.at[idx_v.at[pl.ds(c * _CHUNK, _CHUNK)]],
          rows_v.at[buf],
          gsems[buf],
      )

    def wait_gather(buf):
      pltpu.make_async_copy(
          table_hbm.at[idx_v.at[pl.ds(0, _CHUNK)]],
          rows_v.at[buf],
          gsems[buf],
      ).wait()

    def start_out(c, buf):
      pltpu.async_copy(
          rows_v.at[buf],
          out_hbm.at[pl.ds(base + c * _CHUNK, _CHUNK)],
          osems[buf],
      )

    def wait_out(buf):
      pltpu.make_async_copy(
          rows_v.at[buf],
          out_hbm.at[pl.ds(base, _CHUNK)],
          osems[buf],
      ).wait()

    start_gather(0, 0)
    start_gather(1, 1)

    @pl.loop(0, n_chunks, step=2)
    def _(g):
      for buf in range(2):
        c = g + buf
        wait_gather(buf)
        start_out(c, buf)
        wait_out(buf)

        @pl.when(c + 2 < n_chunks)
        def _():
          start_gather(c + 2, buf)

  return gather_kernel


def kernel(x, table):
  b = x.shape[0] * x.shape[1]
  d = table.shape[1]
  idx = x.reshape((b,)).astype(jnp.int32)
  out = _make_gather(b, d)(table, idx)
  return out.reshape(x.shape + (d,))
